# Initial kernel scaffold; baseline (speedup 1.0000x reference)
import jax
import jax.numpy as jnp
from jax.experimental import pallas as pl


def kernel(state, action, expert_states, expert_actions):
    raise NotImplementedError("write your pallas kernel here")


# speedup vs baseline: 1.2180x; 1.2180x over previous
import math

import jax
import jax.numpy as jnp
from jax.experimental import pallas as pl
from jax.experimental.pallas import tpu as pltpu

_K = 50000
_DS = 256
_DA = 64
_BK = 2000
_NB = _K // _BK
_TH = 1000
_RS = 5.0
_BW = 5.0 * _TH / math.sqrt(_DS + _DA)
_EW = 1.0 / _K
_UNITS = (1.0 / _TH - 1e-6) * _K
_ROUNDS = 50


def _stats_body(es_ref, ea_ref, ss_ref, sq_ref, as_ref, aq_ref):
    i = pl.program_id(0)

    @pl.when(i == 0)
    def _():
        ss_ref[...] = jnp.zeros_like(ss_ref)
        sq_ref[...] = jnp.zeros_like(sq_ref)
        as_ref[...] = jnp.zeros_like(as_ref)
        aq_ref[...] = jnp.zeros_like(aq_ref)

    es = es_ref[...]
    ea = ea_ref[...]
    ss_ref[...] += jnp.sum(es, axis=0, keepdims=True)
    sq_ref[...] += jnp.sum(es * es, axis=0, keepdims=True)
    as_ref[...] += jnp.sum(ea, axis=0, keepdims=True)
    aq_ref[...] += jnp.sum(ea * ea, axis=0, keepdims=True)


def _dist_body(es_ref, ea_ref, s_ref, a_ref, ss_ref, sq_ref, as_ref, aq_ref,
               d_ref):
    n = jnp.float32(_K)
    ms = ss_ref[...] / n
    vs = jnp.maximum(sq_ref[...] / n - ms * ms, 0.0)
    inv_s = 1.0 / (jnp.sqrt(vs) + 1e-8)
    ma = as_ref[...] / n
    va = jnp.maximum(aq_ref[...] / n - ma * ma, 0.0)
    inv_a = 1.0 / (jnp.sqrt(va) + 1e-8)
    ds = (es_ref[...] - s_ref[...]) * inv_s
    da = (ea_ref[...] - a_ref[...]) * inv_a
    d2 = (jnp.sum(ds * ds, axis=1, keepdims=True)
          + jnp.sum(da * da, axis=1, keepdims=True))
    d_ref[...] = jnp.sqrt(d2)


def _select_body(d_ref, r_ref, ds_ref):
    ds_ref[...] = d_ref[...]

    def round_fn(_, carry):
        rem, cost = carry
        d = ds_ref[...]
        m = jnp.min(d)
        cnt = jnp.sum(jnp.where(d == m, 1.0, 0.0))
        use = jnp.minimum(cnt, rem)
        cost = cost + jnp.where(use > 0, use * m, 0.0)
        ds_ref[...] = jnp.where(d == m, jnp.inf, d)
        return rem - use, cost

    _, cost = jax.lax.fori_loop(
        0, _ROUNDS, round_fn, (jnp.float32(_UNITS), jnp.float32(0.0)))
    reward = jnp.float32(_RS) * jnp.exp(
        jnp.float32(-_BW) * (cost * jnp.float32(_EW)))
    r_ref[...] = jnp.reshape(reward, (1, 1))


def kernel(state, action, expert_states, expert_actions):
    f32 = jnp.float32
    ss, sq, as_, aq = pl.pallas_call(
        _stats_body,
        grid=(_NB,),
        in_specs=[
            pl.BlockSpec((_BK, _DS), lambda i: (i, 0)),
            pl.BlockSpec((_BK, _DA), lambda i: (i, 0)),
        ],
        out_specs=[
            pl.BlockSpec((1, _DS), lambda i: (0, 0)),
            pl.BlockSpec((1, _DS), lambda i: (0, 0)),
            pl.BlockSpec((1, _DA), lambda i: (0, 0)),
            pl.BlockSpec((1, _DA), lambda i: (0, 0)),
        ],
        out_shape=[
            jax.ShapeDtypeStruct((1, _DS), f32),
            jax.ShapeDtypeStruct((1, _DS), f32),
            jax.ShapeDtypeStruct((1, _DA), f32),
            jax.ShapeDtypeStruct((1, _DA), f32),
        ],
    )(expert_states, expert_actions)

    d = pl.pallas_call(
        _dist_body,
        grid=(_NB,),
        in_specs=[
            pl.BlockSpec((_BK, _DS), lambda i: (i, 0)),
            pl.BlockSpec((_BK, _DA), lambda i: (i, 0)),
            pl.BlockSpec((1, _DS), lambda i: (0, 0)),
            pl.BlockSpec((1, _DA), lambda i: (0, 0)),
            pl.BlockSpec((1, _DS), lambda i: (0, 0)),
            pl.BlockSpec((1, _DS), lambda i: (0, 0)),
            pl.BlockSpec((1, _DA), lambda i: (0, 0)),
            pl.BlockSpec((1, _DA), lambda i: (0, 0)),
        ],
        out_specs=pl.BlockSpec((_BK, 1), lambda i: (i, 0)),
        out_shape=jax.ShapeDtypeStruct((_K, 1), f32),
    )(expert_states, expert_actions, state, action, ss, sq, as_, aq)

    r = pl.pallas_call(
        _select_body,
        out_shape=jax.ShapeDtypeStruct((1, 1), f32),
        scratch_shapes=[pltpu.VMEM((_NB, _BK), f32)],
    )(d.reshape(_NB, _BK))
    return r[0, 0]
